# baseline (speedup 1.0000x reference)
import functools

import jax
import jax.numpy as jnp
from jax import lax
from jax.experimental import pallas as pl
from jax.experimental.pallas import tpu as pltpu
from jax.experimental.pallas import tpu_sc as plsc

N_NODES = 10000
N_EDGES = 320000
D = 128

NC = 2
NS = 16
NW = NC * NS
E_W = N_EDGES // NW
CHUNK = 128
N_CHUNKS = E_W // CHUNK
TAIL = E_W - N_CHUNKS * CHUNK
K = 2
M = 26
N_BODIES = N_CHUNKS // M
N_PAD = 10240
ROWS_PER_TILE = N_PAD // NS


def _sc_scatter(edges, senders, receivers, zeros_block):
    mesh = plsc.VectorSubcoreMesh(core_axis_name="c", subcore_axis_name="s")

    @functools.partial(
        pl.kernel,
        mesh=mesh,
        out_type=jax.ShapeDtypeStruct((NC, N_PAD, D), jnp.float32),
        scratch_types=[
            pltpu.VMEM((K, CHUNK, D), jnp.float32),
        ] + [pltpu.VMEM((CHUNK,), jnp.int32) for _ in range(2 * K)] + [
            pltpu.VMEM((TAIL, D), jnp.float32),
            pltpu.VMEM((TAIL,), jnp.int32),
            pltpu.VMEM((TAIL,), jnp.int32),
            pltpu.VMEM_SHARED((N_PAD, D), jnp.float32),
            pltpu.SemaphoreType.DMA((K,)),
            pltpu.SemaphoreType.DMA((K,)),
            pltpu.SemaphoreType.DMA,
            pltpu.SemaphoreType.DMA,
        ],
    )
    def k(edges_hbm, send_hbm, recv_hbm, zeros_hbm, out_hbm,
          rows_v, *rest):
        idx_bufs, rest = rest[:2 * K], rest[2 * K:]
        (rows_t, sidx_t, ridx_t, acc_sh,
         load_sem, idx_sem, scat_sem, tail_sem) = rest
        c = lax.axis_index("c")
        s = lax.axis_index("s")
        wid = s * NC + c
        tile_base = s * ROWS_PER_TILE
        zero = pltpu.async_copy(
            zeros_hbm, acc_sh.at[pl.ds(tile_base, ROWS_PER_TILE)], scat_sem)
        ebase = wid * E_W
        toff = ebase + N_CHUNKS * CHUNK
        tail_loads = (
            pltpu.async_copy(edges_hbm.at[pl.ds(toff, TAIL)], rows_t,
                             tail_sem),
            pltpu.async_copy(send_hbm.at[pl.ds(toff, TAIL)], sidx_t,
                             tail_sem),
            pltpu.async_copy(recv_hbm.at[pl.ds(toff, TAIL)], ridx_t,
                             tail_sem),
        )

        def fire_load(base, j):
            off = base + j * CHUNK
            slot = j % K
            return (
                pltpu.async_copy(edges_hbm.at[pl.ds(off, CHUNK)],
                                 rows_v.at[slot], load_sem.at[slot]),
                pltpu.async_copy(send_hbm.at[pl.ds(off, CHUNK)],
                                 idx_bufs[2 * slot], idx_sem.at[slot]),
                pltpu.async_copy(recv_hbm.at[pl.ds(off, CHUNK)],
                                 idx_bufs[2 * slot + 1], idx_sem.at[slot]),
            )

        def do_chunks(i, n_chunks, pre_barrier=None):
            base = ebase + i * M * CHUNK
            loads = {}
            scats = {}
            for j in range(min(K, n_chunks)):
                loads[j] = fire_load(base, j)
            if pre_barrier is not None:
                pre_barrier()
            for j in range(n_chunks):
                prev = j - 1
                if prev >= 0 and prev + K < n_chunks:
                    for d in scats[prev]:
                        d.wait()
                    loads[prev + K] = fire_load(base, prev + K)
                for d in loads[j]:
                    d.wait()
                slot = j % K
                scats[j] = (
                    pltpu.async_copy(rows_v.at[slot],
                                     acc_sh.at[idx_bufs[2 * slot]],
                                     scat_sem, add=True),
                    pltpu.async_copy(rows_v.at[slot],
                                     acc_sh.at[idx_bufs[2 * slot + 1]],
                                     scat_sem, add=True),
                )
            for j in range(max(0, n_chunks - K), n_chunks):
                for d in scats[j]:
                    d.wait()

        def body(i, carry):
            do_chunks(i, M)
            return carry

        def first_barrier():
            zero.wait()
            plsc.subcore_barrier()

        do_chunks(0, M, pre_barrier=first_barrier)
        lax.fori_loop(1, N_BODIES, body, 0)
        for d in tail_loads:
            d.wait()
        pltpu.sync_copy(rows_t, acc_sh.at[sidx_t], add=True)
        pltpu.sync_copy(rows_t, acc_sh.at[ridx_t], add=True)
        plsc.subcore_barrier()
        pltpu.sync_copy(acc_sh.at[pl.ds(tile_base, ROWS_PER_TILE)],
                        out_hbm.at[c, pl.ds(tile_base, ROWS_PER_TILE)])

    return k(edges, senders, receivers, zeros_block)


def _tc_mlp(acc, nodes, W, b2d):
    BN = 2000

    def body(a0, a1, n, w, b, o):
        wv = w[...]
        agg = a0[0] + a1[0]
        o[...] = (
            jnp.dot(agg, wv[:D], preferred_element_type=jnp.float32)
            + jnp.dot(n[...], wv[D:], preferred_element_type=jnp.float32)
            + b[...]
        )

    return pl.pallas_call(
        body,
        grid=(N_NODES // BN,),
        in_specs=[
            pl.BlockSpec((1, BN, D), lambda i: (0, i, 0)),
            pl.BlockSpec((1, BN, D), lambda i: (1, i, 0)),
            pl.BlockSpec((BN, D), lambda i: (i, 0)),
            pl.BlockSpec((2 * D, D), lambda i: (0, 0)),
            pl.BlockSpec((1, D), lambda i: (0, 0)),
        ],
        out_specs=pl.BlockSpec((BN, D), lambda i: (i, 0)),
        out_shape=jax.ShapeDtypeStruct((N_NODES, D), jnp.float32),
    )(acc, acc, nodes, W, b2d)


def kernel(nodes, edges, senders, receivers, W, b):
    zeros_block = jnp.zeros((ROWS_PER_TILE, D), jnp.float32)
    acc = _sc_scatter(edges, senders.astype(jnp.int32),
                      receivers.astype(jnp.int32), zeros_block)
    return _tc_mlp(acc, nodes, W, b.reshape(1, D))


# speedup vs baseline: 1.0103x; 1.0103x over previous
import functools

import jax
import jax.numpy as jnp
from jax import lax
from jax.experimental import pallas as pl
from jax.experimental.pallas import tpu as pltpu
from jax.experimental.pallas import tpu_sc as plsc

N_NODES = 10000
N_EDGES = 320000
D = 128

NC = 2
NS = 16
NW = NC * NS
E_W = N_EDGES // NW
CHUNK = 80
N_CHUNKS = E_W // CHUNK
K = 4
M1 = 63
M2 = 62
N_PAD = 10240
ROWS_PER_TILE = N_PAD // NS


def _sc_scatter(edges, senders, receivers, zeros_block):
    mesh = plsc.VectorSubcoreMesh(core_axis_name="c", subcore_axis_name="s")

    @functools.partial(
        pl.kernel,
        mesh=mesh,
        out_type=jax.ShapeDtypeStruct((NC, N_PAD, D), jnp.float32),
        scratch_types=[
            pltpu.VMEM((K, CHUNK, D), jnp.float32),
        ] + [pltpu.VMEM((CHUNK,), jnp.int32) for _ in range(2 * K)] + [
            pltpu.VMEM_SHARED((N_PAD, D), jnp.float32),
            pltpu.SemaphoreType.DMA((K,)),
            pltpu.SemaphoreType.DMA((K,)),
            pltpu.SemaphoreType.DMA,
        ],
    )
    def k(edges_hbm, send_hbm, recv_hbm, zeros_hbm, out_hbm,
          rows_v, *rest):
        idx_bufs, rest = rest[:2 * K], rest[2 * K:]
        acc_sh, load_sem, idx_sem, scat_sem = rest
        c = lax.axis_index("c")
        s = lax.axis_index("s")
        wid = s * NC + c
        tile_base = s * ROWS_PER_TILE
        zero = pltpu.async_copy(
            zeros_hbm, acc_sh.at[pl.ds(tile_base, ROWS_PER_TILE)], scat_sem)
        ebase = wid * E_W

        def fire_load(base, j):
            off = base + j * CHUNK
            slot = j % K
            return (
                pltpu.async_copy(edges_hbm.at[pl.ds(off, CHUNK)],
                                 rows_v.at[slot], load_sem.at[slot]),
                pltpu.async_copy(send_hbm.at[pl.ds(off, CHUNK)],
                                 idx_bufs[2 * slot], idx_sem.at[slot]),
                pltpu.async_copy(recv_hbm.at[pl.ds(off, CHUNK)],
                                 idx_bufs[2 * slot + 1], idx_sem.at[slot]),
            )

        def do_chunks(base, n_chunks, pre_barrier=None):
            loads = {}
            scats = {}
            for j in range(min(K, n_chunks)):
                loads[j] = fire_load(base, j)
            if pre_barrier is not None:
                pre_barrier()
            for j in range(n_chunks):
                prev = j - 1
                if prev >= 0 and prev + K < n_chunks:
                    for d in scats[prev]:
                        d.wait()
                    loads[prev + K] = fire_load(base, prev + K)
                for d in loads[j]:
                    d.wait()
                slot = j % K
                scats[j] = (
                    pltpu.async_copy(rows_v.at[slot],
                                     acc_sh.at[idx_bufs[2 * slot]],
                                     scat_sem, add=True),
                    pltpu.async_copy(rows_v.at[slot],
                                     acc_sh.at[idx_bufs[2 * slot + 1]],
                                     scat_sem, add=True),
                )
            for j in range(max(0, n_chunks - K), n_chunks):
                for d in scats[j]:
                    d.wait()

        def first_barrier():
            zero.wait()
            plsc.subcore_barrier()

        do_chunks(ebase, M1, pre_barrier=first_barrier)
        do_chunks(ebase + M1 * CHUNK, M2)
        plsc.subcore_barrier()
        pltpu.sync_copy(acc_sh.at[pl.ds(tile_base, ROWS_PER_TILE)],
                        out_hbm.at[c, pl.ds(tile_base, ROWS_PER_TILE)])

    return k(edges, senders, receivers, zeros_block)


def _tc_mlp(acc, nodes, W, b2d):
    BN = 2000

    def body(a0, a1, n, w, b, o):
        wv = w[...]
        agg = a0[0] + a1[0]
        o[...] = (
            jnp.dot(agg, wv[:D], preferred_element_type=jnp.float32)
            + jnp.dot(n[...], wv[D:], preferred_element_type=jnp.float32)
            + b[...]
        )

    return pl.pallas_call(
        body,
        grid=(N_NODES // BN,),
        in_specs=[
            pl.BlockSpec((1, BN, D), lambda i: (0, i, 0)),
            pl.BlockSpec((1, BN, D), lambda i: (1, i, 0)),
            pl.BlockSpec((BN, D), lambda i: (i, 0)),
            pl.BlockSpec((2 * D, D), lambda i: (0, 0)),
            pl.BlockSpec((1, D), lambda i: (0, 0)),
        ],
        out_specs=pl.BlockSpec((BN, D), lambda i: (i, 0)),
        out_shape=jax.ShapeDtypeStruct((N_NODES, D), jnp.float32),
    )(acc, acc, nodes, W, b2d)


def kernel(nodes, edges, senders, receivers, W, b):
    zeros_block = jnp.zeros((ROWS_PER_TILE, D), jnp.float32)
    acc = _sc_scatter(edges, senders.astype(jnp.int32),
                      receivers.astype(jnp.int32), zeros_block)
    return _tc_mlp(acc, nodes, W, b.reshape(1, D))
